# baseline (speedup 1.0000x reference)
import functools

import jax
import jax.numpy as jnp
from jax import lax
from jax.experimental import pallas as pl
from jax.experimental.pallas import tpu as pltpu
from jax.experimental.pallas import tpu_sc as plsc

N = 10000
E = 320000
D = 128
HALF = 64
ROWS = 3 * N
TAB = 2 * ROWS
TABP = TAB + 8
CHUNK = 64
IDXB = 4

PW = 655360
PU = 327680
WTD_PER_TILE = PW // 16
UNW_PER_TILE = PU // 16
WTD_CHUNKS = WTD_PER_TILE // CHUNK
UNW_CHUNKS = UNW_PER_TILE // CHUNK

DEG_W = PW // 32
DEG_U = PU // 32


def _deg_body(cw_hbm, ww_hbm, cu_hbm, wu_hbm, out_hbm, cbuf, wbuf, dbuf, dacc):
    c = lax.axis_index("c")
    s = lax.axis_index("s")
    wid = c * 16 + s
    zero16 = jnp.zeros((16,), jnp.float32)

    def zfill(i, carry):
        dbuf[i, pl.ds(0, 16)] = zero16
        return carry

    lax.fori_loop(0, CHUNK, zfill, 0)

    def zcopy(b, carry):
        pltpu.sync_copy(dbuf, dacc.at[pl.ds(s * 1875 + b * CHUNK, CHUNK)])
        return carry

    lax.fori_loop(0, 29, zcopy, 0)
    pltpu.sync_copy(dbuf.at[pl.ds(0, 59)],
                    dacc.at[pl.ds(s * 1875 + 29 * CHUNK, 59)])
    plsc.subcore_barrier()

    def region(col_hbm, w_hbm, base, nchunks):
        def chunk(j, carry):
            jj = j % IDXB

            @pl.when(jj == 0)
            def _():
                cb = base + j
                pltpu.sync_copy(col_hbm.at[pl.ds(cb, IDXB)], cbuf)
                pltpu.sync_copy(w_hbm.at[pl.ds(cb, IDXB)], wbuf)

            def grp(g, carry2):
                wv = wbuf[jj, pl.ds(g * 16, 16)]
                for j16 in range(16):
                    bv = jnp.full((16,), wv[j16], jnp.float32)
                    dbuf[g * 16 + j16, pl.ds(0, 16)] = bv
                return carry2

            lax.fori_loop(0, CHUNK // 16, grp, 0)
            pltpu.sync_copy(dbuf, dacc.at[cbuf.at[jj]], add=True)
            return carry

        lax.fori_loop(0, nchunks, chunk, 0)

    region(cw_hbm, ww_hbm, wid * (DEG_W // CHUNK), DEG_W // CHUNK)
    region(cu_hbm, wu_hbm, wid * (DEG_U // CHUNK), DEG_U // CHUNK)
    plsc.subcore_barrier()
    pltpu.sync_copy(dacc.at[pl.ds(s * 1875, 1875)],
                    out_hbm.at[c, pl.ds(s * 1875, 1875)])


def _spmm_body(rw_hbm, cw_hbm, ww_hbm, ru_hbm, cu_hbm, gtab_hbm, out_hbm,
               rbuf, cbuf, wbuf, gbufA, gbufB, acc, gsA, gsB, ssA, ssB):
    c = lax.axis_index("c")
    s = lax.axis_index("s")
    zero16 = jnp.zeros((16,), jnp.float32)

    def zfill(i, carry):
        for q in range(HALF // 16):
            gbufA[i, pl.ds(q * 16, 16)] = zero16
        return carry

    lax.fori_loop(0, CHUNK, zfill, 0)

    def zcopy(b, carry):
        pltpu.sync_copy(gbufA, acc.at[pl.ds(s * 1875 + b * CHUNK, CHUNK)])
        return carry

    lax.fori_loop(0, 29, zcopy, 0)
    pltpu.sync_copy(gbufA.at[pl.ds(0, 59)],
                    acc.at[pl.ds(s * 1875 + 29 * CHUNK, 59)])
    plsc.subcore_barrier()

    def scale(gbuf, wrow):
        def grp(g, carry2):
            wv = wbuf[wrow, pl.ds(g * 16, 16)]
            for j16 in range(16):
                bv = jnp.full((16,), wv[j16], jnp.float32)
                e = g * 16 + j16
                for q in range(HALF // 16):
                    v = gbuf[e, pl.ds(q * 16, 16)]
                    gbuf[e, pl.ds(q * 16, 16)] = v * bv
            return carry2

        lax.fori_loop(0, CHUNK // 16, grp, 0)

    def wait_gather(gbuf, sem):
        pltpu.make_async_copy(gtab_hbm.at[rbuf.at[0]], gbuf, sem).wait()

    def wait_scatter(gbuf, sem):
        pltpu.make_async_copy(gbuf, acc.at[cbuf.at[0]], sem).wait()

    def region(rh, ch, wh, base_row, pairs, weighted):
        def pair(j, carry):
            la = (2 * j) % IDXB

            @pl.when(j > 0)
            def _():
                wait_scatter(gbufB, ssB)

            @pl.when(la == 0)
            def _():
                row0 = base_row + 2 * j
                pltpu.sync_copy(rh.at[pl.ds(row0, IDXB)], rbuf)
                pltpu.sync_copy(ch.at[pl.ds(row0, IDXB)], cbuf)
                if weighted:
                    pltpu.sync_copy(wh.at[pl.ds(row0, IDXB)], wbuf)
                pltpu.async_copy(gtab_hbm.at[rbuf.at[la]], gbufA, gsA)

            wait_gather(gbufA, gsA)
            pltpu.async_copy(gtab_hbm.at[rbuf.at[la + 1]], gbufB, gsB)
            if weighted:
                scale(gbufA, la)
            pltpu.async_copy(gbufA, acc.at[cbuf.at[la]], ssA, add=True)

            wait_gather(gbufB, gsB)
            wait_scatter(gbufA, ssA)

            @pl.when((la == 0) & (j < pairs - 1))
            def _():
                pltpu.async_copy(gtab_hbm.at[rbuf.at[2]], gbufA, gsA)

            if weighted:
                scale(gbufB, la + 1)
            pltpu.async_copy(gbufB, acc.at[cbuf.at[la + 1]], ssB, add=True)
            return carry

        lax.fori_loop(0, pairs, pair, 0)
        wait_scatter(gbufB, ssB)

    region(rw_hbm.at[c], cw_hbm, ww_hbm, s * WTD_CHUNKS, WTD_CHUNKS // 2, True)
    region(ru_hbm.at[c], cu_hbm, None, s * UNW_CHUNKS, UNW_CHUNKS // 2, False)
    plsc.subcore_barrier()
    pltpu.sync_copy(acc.at[pl.ds(s * 1875, 1875)],
                    out_hbm.at[c, pl.ds(s * 1875, 1875)])


@functools.cache
def _sc_kernels():
    mesh = plsc.VectorSubcoreMesh(core_axis_name="c", subcore_axis_name="s",
                                  num_cores=2, num_subcores=16)
    params = pltpu.CompilerParams(use_tc_tiling_on_sc=False)
    deg = pl.kernel(
        _deg_body,
        out_type=jax.ShapeDtypeStruct((2, ROWS, 16), jnp.float32),
        mesh=mesh,
        compiler_params=params,
        scratch_types=[
            pltpu.VMEM((IDXB, CHUNK), jnp.int32),
            pltpu.VMEM((IDXB, CHUNK), jnp.float32),
            pltpu.VMEM((CHUNK, 16), jnp.float32),
            pltpu.VMEM_SHARED((ROWS, 16), jnp.float32),
        ],
    )
    spmm = pl.kernel(
        _spmm_body,
        out_type=jax.ShapeDtypeStruct((2, ROWS, HALF), jnp.float32),
        mesh=mesh,
        compiler_params=params,
        scratch_types=[
            pltpu.VMEM((IDXB, CHUNK), jnp.int32),
            pltpu.VMEM((IDXB, CHUNK), jnp.int32),
            pltpu.VMEM((IDXB, CHUNK), jnp.float32),
            pltpu.VMEM((CHUNK, HALF), jnp.float32),
            pltpu.VMEM((CHUNK, HALF), jnp.float32),
            pltpu.VMEM_SHARED((ROWS, HALF), jnp.float32),
            pltpu.SemaphoreType.DMA,
            pltpu.SemaphoreType.DMA,
            pltpu.SemaphoreType.DMA,
            pltpu.SemaphoreType.DMA,
        ],
    )
    return deg, spmm


_R = 1000


def _tc1_body(x_ref, w1_ref, degs_ref, g_ref, dinv_ref):
    deg = jnp.sum(degs_ref[...], axis=2) + 1.0
    dinv = jnp.where(deg > 0, lax.rsqrt(deg), 0.0)
    dinv_ref[...] = dinv
    h = jnp.dot(x_ref[...], w1_ref[...].T, preferred_element_type=jnp.float32)
    for k in range(3):
        gk = h * dinv[:, k][:, None]
        g_ref[0, k] = gk[:, :HALF]
        g_ref[1, k] = gk[:, HALF:]


def _tc_mid_parts(acc_ref, g_ref, dinv_ref, b_ref):
    parts = []
    dinv = dinv_ref[...]
    for k in range(3):
        lo = acc_ref[0, k] + g_ref[0, k]
        hi = acc_ref[1, k] + g_ref[1, k]
        full = jnp.concatenate([lo, hi], axis=1)
        xk = dinv[:, k][:, None] * full + b_ref[...]
        parts.append(jax.nn.relu(xk))
    return jnp.concatenate(parts, axis=1)


def _tc2_body(acc_ref, g_ref, dinv_ref, b1_ref, w2_ref, g2_ref):
    xcat = _tc_mid_parts(acc_ref, g_ref, dinv_ref, b1_ref)
    h2 = jnp.dot(xcat, w2_ref[...].T, preferred_element_type=jnp.float32)
    dinv = dinv_ref[...]
    for k in range(3):
        g2_ref[0, k] = h2[:, :HALF] * dinv[:, k][:, None]
        g2_ref[1, k] = h2[:, HALF:] * dinv[:, k][:, None]


def _tc3_body(acc_ref, g2_ref, dinv_ref, b2_ref, wc_ref, bc_ref, out_ref):
    xcat = _tc_mid_parts(acc_ref, g2_ref, dinv_ref, b2_ref)
    logits = jnp.dot(xcat, wc_ref[...].T, preferred_element_type=jnp.float32)
    logits = logits + bc_ref[...]
    m = jnp.max(logits, axis=1, keepdims=True)
    lse = jnp.log(jnp.sum(jnp.exp(logits - m), axis=1, keepdims=True)) + m
    out_ref[...] = logits - lse


def _tc1(x, W1, degs):
    return pl.pallas_call(
        _tc1_body,
        grid=(N // _R,),
        in_specs=[
            pl.BlockSpec((_R, D), lambda i: (i, 0)),
            pl.BlockSpec((D, D), lambda i: (0, 0)),
            pl.BlockSpec((_R, 3, 2), lambda i: (i, 0, 0)),
        ],
        out_specs=[
            pl.BlockSpec((2, 3, _R, HALF), lambda i: (0, 0, i, 0)),
            pl.BlockSpec((_R, 3), lambda i: (i, 0)),
        ],
        out_shape=[
            jax.ShapeDtypeStruct((2, 3, N, HALF), jnp.float32),
            jax.ShapeDtypeStruct((N, 3), jnp.float32),
        ],
    )(x, W1, degs)


def _tc2(acc, g, dinv, b1, W2):
    return pl.pallas_call(
        _tc2_body,
        grid=(N // _R,),
        in_specs=[
            pl.BlockSpec((2, 3, _R, HALF), lambda i: (0, 0, i, 0)),
            pl.BlockSpec((2, 3, _R, HALF), lambda i: (0, 0, i, 0)),
            pl.BlockSpec((_R, 3), lambda i: (i, 0)),
            pl.BlockSpec((1, D), lambda i: (0, 0)),
            pl.BlockSpec((D, 3 * D), lambda i: (0, 0)),
        ],
        out_specs=pl.BlockSpec((2, 3, _R, HALF), lambda i: (0, 0, i, 0)),
        out_shape=jax.ShapeDtypeStruct((2, 3, N, HALF), jnp.float32),
    )(acc, g, dinv, b1, W2)


def _tc3(acc, g2, dinv, b2, Wc, bc):
    return pl.pallas_call(
        _tc3_body,
        grid=(N // _R,),
        in_specs=[
            pl.BlockSpec((2, 3, _R, HALF), lambda i: (0, 0, i, 0)),
            pl.BlockSpec((2, 3, _R, HALF), lambda i: (0, 0, i, 0)),
            pl.BlockSpec((_R, 3), lambda i: (i, 0)),
            pl.BlockSpec((1, D), lambda i: (0, 0)),
            pl.BlockSpec((HALF, 3 * D), lambda i: (0, 0)),
            pl.BlockSpec((1, HALF), lambda i: (0, 0)),
        ],
        out_specs=pl.BlockSpec((_R, HALF), lambda i: (i, 0)),
        out_shape=jax.ShapeDtypeStruct((N, HALF), jnp.float32),
    )(acc, g2, dinv, b2, Wc, bc.reshape(1, HALF))


def kernel(x, edge_index, edge_in, edge_out, in_w, out_w, W1, b1, W2, b2, Wc, bc):
    ei = edge_index.astype(jnp.int32)
    ein = edge_in.astype(jnp.int32)
    eout = edge_out.astype(jnp.int32)

    padw = PW - 2 * E
    padu = PU - E
    rw = jnp.concatenate([ein[0] + N, eout[0] + 2 * N])
    cw = jnp.concatenate([ein[1] + N, eout[1] + 2 * N,
                          jnp.zeros((padw,), jnp.int32)])
    ww = jnp.concatenate([in_w.astype(jnp.float32), out_w.astype(jnp.float32),
                          jnp.zeros((padw,), jnp.float32)])
    ru = ei[0]
    cu = jnp.concatenate([ei[1], jnp.zeros((padu,), jnp.int32)])
    wu = jnp.concatenate([jnp.ones((E,), jnp.float32),
                          jnp.zeros((padu,), jnp.float32)])
    zpadw = jnp.full((padw,), TAB, jnp.int32)
    zpadu = jnp.full((padu,), TAB, jnp.int32)
    rw2 = jnp.stack([jnp.concatenate([rw, zpadw]),
                     jnp.concatenate([rw + ROWS, zpadw])])
    ru2 = jnp.stack([jnp.concatenate([ru, zpadu]),
                     jnp.concatenate([ru + ROWS, zpadu])])
    rw2 = rw2.reshape(2, PW // CHUNK, CHUNK)
    ru2 = ru2.reshape(2, PU // CHUNK, CHUNK)
    cw = cw.reshape(PW // CHUNK, CHUNK)
    ww = ww.reshape(PW // CHUNK, CHUNK)
    cu = cu.reshape(PU // CHUNK, CHUNK)
    wu = wu.reshape(PU // CHUNK, CHUNK)

    _deg_kernel, _spmm_kernel = _sc_kernels()
    dd = _deg_kernel(cw, ww, cu, wu)
    degs = dd[:, :, 0].reshape(2, 3, N).transpose(2, 1, 0)

    g, dinv = _tc1(x, W1, degs)
    def table(garr):
        return jnp.concatenate([garr.reshape(TAB, HALF),
                                jnp.zeros((TABP - TAB, HALF), jnp.float32)])

    acc1 = _spmm_kernel(rw2, cw, ww, ru2, cu, table(g))
    acc1 = acc1.reshape(2, 3, N, HALF)

    g2 = _tc2(acc1, g, dinv, b1, W2)
    acc2 = _spmm_kernel(rw2, cw, ww, ru2, cu, table(g2))
    acc2 = acc2.reshape(2, 3, N, HALF)

    return _tc3(acc2, g2, dinv, b2, Wc, bc)


# speedup vs baseline: 1.4233x; 1.4233x over previous
import functools

import jax
import jax.numpy as jnp
from jax import lax
from jax.experimental import pallas as pl
from jax.experimental.pallas import tpu as pltpu
from jax.experimental.pallas import tpu_sc as plsc

N = 10000
E = 320000
D = 128
HALF = 64
ROWS = 3 * N
TAB = 2 * ROWS
TABP = TAB + 8
CHUNK = 128
IDXB = 8
DCH = 64
DIDXB = 4

PW = 655360
PU = 327680
DEG_W = PW // 32
DEG_U = PU // 32

EP = 327680
SET_CHUNKS = EP // CHUNK
P1_CHUNKS = SET_CHUNKS // 16
P2_CHUNKS = SET_CHUNKS // 32


def _deg_body(cw_hbm, ww_hbm, cu_hbm, wu_hbm, out_hbm, cbuf, wbuf, dbuf, dacc):
    c = lax.axis_index("c")
    s = lax.axis_index("s")
    wid = c * 16 + s
    zero16 = jnp.zeros((16,), jnp.float32)

    def zfill(i, carry):
        dbuf[i, pl.ds(0, 16)] = zero16
        return carry

    lax.fori_loop(0, DCH, zfill, 0)

    def zcopy(b, carry):
        pltpu.sync_copy(dbuf, dacc.at[pl.ds(s * 1875 + b * DCH, DCH)])
        return carry

    lax.fori_loop(0, 29, zcopy, 0)
    pltpu.sync_copy(dbuf.at[pl.ds(0, 59)],
                    dacc.at[pl.ds(s * 1875 + 29 * DCH, 59)])
    plsc.subcore_barrier()

    def region(col_hbm, w_hbm, base, nchunks):
        def chunk(j, carry):
            jj = j % DIDXB

            @pl.when(jj == 0)
            def _():
                cb = base + j
                pltpu.sync_copy(col_hbm.at[pl.ds(cb, DIDXB)], cbuf)
                pltpu.sync_copy(w_hbm.at[pl.ds(cb, DIDXB)], wbuf)

            def grp(g, carry2):
                wv = wbuf[jj, pl.ds(g * 16, 16)]
                for j16 in range(16):
                    bv = jnp.full((16,), wv[j16], jnp.float32)
                    dbuf[g * 16 + j16, pl.ds(0, 16)] = bv
                return carry2

            lax.fori_loop(0, DCH // 16, grp, 0)
            pltpu.sync_copy(dbuf, dacc.at[cbuf.at[jj]], add=True)
            return carry

        lax.fori_loop(0, nchunks, chunk, 0)

    region(cw_hbm, ww_hbm, wid * (DEG_W // DCH), DEG_W // DCH)
    region(cu_hbm, wu_hbm, wid * (DEG_U // DCH), DEG_U // DCH)
    plsc.subcore_barrier()
    pltpu.sync_copy(dacc.at[pl.ds(s * 1875, 1875)],
                    out_hbm.at[c, pl.ds(s * 1875, 1875)])


def _spmm_body(r0, c0, r1, c1, w1, r2, c2, w2, gtab_hbm, out_hbm,
               rbuf, cbuf, wbuf, gbufA, gbufB, acc, gsA, gsB, ssA, ssB):
    c = lax.axis_index("c")
    s = lax.axis_index("s")
    zero16 = jnp.zeros((16,), jnp.float32)

    def zfill(i, carry):
        for q in range(D // 16):
            gbufA[i, pl.ds(q * 16, 16)] = zero16
        return carry

    def zero_acc():
        lax.fori_loop(0, CHUNK, zfill, 0)

        def zcopy(b, carry):
            pltpu.sync_copy(gbufA, acc.at[pl.ds(s * 625 + b * CHUNK, CHUNK)])
            return carry

        lax.fori_loop(0, 4, zcopy, 0)
        pltpu.sync_copy(gbufA.at[pl.ds(0, 113)],
                        acc.at[pl.ds(s * 625 + 4 * CHUNK, 113)])

    def copy_out(phase):
        pltpu.sync_copy(acc.at[pl.ds(s * 625, 625)],
                        out_hbm.at[c, phase, pl.ds(s * 625, 625)])

    def scale(gbuf, wrow):
        def grp(g, carry2):
            wv = wbuf[wrow, pl.ds(g * 16, 16)]
            for j16 in range(16):
                bv = jnp.full((16,), wv[j16], jnp.float32)
                e = g * 16 + j16
                for q in range(D // 16):
                    v = gbuf[e, pl.ds(q * 16, 16)]
                    gbuf[e, pl.ds(q * 16, 16)] = v * bv
            return carry2

        lax.fori_loop(0, CHUNK // 16, grp, 0)

    def wait_gather(gbuf, sem):
        pltpu.make_async_copy(gtab_hbm.at[rbuf.at[0]], gbuf, sem).wait()

    def wait_scatter(gbuf, sem):
        pltpu.make_async_copy(gbuf, acc.at[cbuf.at[0]], sem).wait()

    def region(rh, ch, wh, base_row, pairs, weighted):
        def pair(j, carry):
            la = (2 * j) % IDXB

            @pl.when(j > 0)
            def _():
                wait_scatter(gbufB, ssB)

            @pl.when(la == 0)
            def _():
                row0 = base_row + 2 * j
                pltpu.sync_copy(rh.at[pl.ds(row0, IDXB)], rbuf)
                pltpu.sync_copy(ch.at[pl.ds(row0, IDXB)], cbuf)
                if weighted:
                    pltpu.sync_copy(wh.at[pl.ds(row0, IDXB)], wbuf)
                pltpu.async_copy(gtab_hbm.at[rbuf.at[la]], gbufA, gsA)

            wait_gather(gbufA, gsA)
            pltpu.async_copy(gtab_hbm.at[rbuf.at[la + 1]], gbufB, gsB)
            if weighted:
                scale(gbufA, la)
            pltpu.async_copy(gbufA, acc.at[cbuf.at[la]], ssA, add=True)

            wait_gather(gbufB, gsB)
            wait_scatter(gbufA, ssA)

            @pl.when((la < IDXB - 2) & (j < pairs - 1))
            def _():
                pltpu.async_copy(gtab_hbm.at[rbuf.at[la + 2]], gbufA, gsA)

            if weighted:
                scale(gbufB, la + 1)
            pltpu.async_copy(gbufB, acc.at[cbuf.at[la + 1]], ssB, add=True)
            return carry

        lax.fori_loop(0, pairs, pair, 0)
        wait_scatter(gbufB, ssB)

    zero_acc()
    plsc.subcore_barrier()

    @pl.when(c == 0)
    def _():
        region(r0, c0, None, s * P1_CHUNKS, P1_CHUNKS // 2, False)

    @pl.when(c == 1)
    def _():
        region(r1, c1, w1, s * P1_CHUNKS, P1_CHUNKS // 2, True)

    plsc.subcore_barrier()
    copy_out(0)
    zero_acc()
    plsc.subcore_barrier()
    region(r2, c2, w2, c * (SET_CHUNKS // 2) + s * P2_CHUNKS,
           P2_CHUNKS // 2, True)
    plsc.subcore_barrier()
    copy_out(1)


@functools.cache
def _sc_kernels():
    mesh = plsc.VectorSubcoreMesh(core_axis_name="c", subcore_axis_name="s",
                                  num_cores=2, num_subcores=16)
    params = pltpu.CompilerParams(use_tc_tiling_on_sc=False)
    deg = pl.kernel(
        _deg_body,
        out_type=jax.ShapeDtypeStruct((2, ROWS, 16), jnp.float32),
        mesh=mesh,
        compiler_params=params,
        scratch_types=[
            pltpu.VMEM((DIDXB, DCH), jnp.int32),
            pltpu.VMEM((DIDXB, DCH), jnp.float32),
            pltpu.VMEM((DCH, 16), jnp.float32),
            pltpu.VMEM_SHARED((ROWS, 16), jnp.float32),
        ],
    )
    spmm = pl.kernel(
        _spmm_body,
        out_type=jax.ShapeDtypeStruct((2, 2, N, D), jnp.float32),
        mesh=mesh,
        compiler_params=params,
        scratch_types=[
            pltpu.VMEM((IDXB, CHUNK), jnp.int32),
            pltpu.VMEM((IDXB, CHUNK), jnp.int32),
            pltpu.VMEM((IDXB, CHUNK), jnp.float32),
            pltpu.VMEM((CHUNK, D), jnp.float32),
            pltpu.VMEM((CHUNK, D), jnp.float32),
            pltpu.VMEM_SHARED((N, D), jnp.float32),
            pltpu.SemaphoreType.DMA,
            pltpu.SemaphoreType.DMA,
            pltpu.SemaphoreType.DMA,
            pltpu.SemaphoreType.DMA,
        ],
    )
    return deg, spmm


_R = 1000


def _tc1_body(x_ref, w1_ref, degs_ref, g_ref, dinv_ref):
    deg = jnp.sum(degs_ref[...], axis=2) + 1.0
    dinv = jnp.where(deg > 0, lax.rsqrt(deg), 0.0)
    dinv_ref[...] = dinv
    h = jnp.dot(x_ref[...], w1_ref[...].T, preferred_element_type=jnp.float32)
    for k in range(3):
        g_ref[k] = h * dinv[:, k][:, None]


def _tc_mid_parts(acc_ref, g_ref, dinv_ref, b_ref):
    accs = [acc_ref[0, 0], acc_ref[1, 0], acc_ref[0, 1] + acc_ref[1, 1]]
    dinv = dinv_ref[...]
    parts = []
    for k in range(3):
        xk = dinv[:, k][:, None] * (accs[k] + g_ref[k]) + b_ref[...]
        parts.append(jax.nn.relu(xk))
    return jnp.concatenate(parts, axis=1)


def _tc2_body(acc_ref, g_ref, dinv_ref, b1_ref, w2_ref, g2_ref):
    xcat = _tc_mid_parts(acc_ref, g_ref, dinv_ref, b1_ref)
    h2 = jnp.dot(xcat, w2_ref[...].T, preferred_element_type=jnp.float32)
    dinv = dinv_ref[...]
    for k in range(3):
        g2_ref[k] = h2 * dinv[:, k][:, None]


def _tc3_body(acc_ref, g2_ref, dinv_ref, b2_ref, wc_ref, bc_ref, out_ref):
    xcat = _tc_mid_parts(acc_ref, g2_ref, dinv_ref, b2_ref)
    logits = jnp.dot(xcat, wc_ref[...].T, preferred_element_type=jnp.float32)
    logits = logits + bc_ref[...]
    m = jnp.max(logits, axis=1, keepdims=True)
    lse = jnp.log(jnp.sum(jnp.exp(logits - m), axis=1, keepdims=True)) + m
    out_ref[...] = logits - lse


def _tc1(x, W1, degs):
    return pl.pallas_call(
        _tc1_body,
        grid=(N // _R,),
        in_specs=[
            pl.BlockSpec((_R, D), lambda i: (i, 0)),
            pl.BlockSpec((D, D), lambda i: (0, 0)),
            pl.BlockSpec((_R, 3, 2), lambda i: (i, 0, 0)),
        ],
        out_specs=[
            pl.BlockSpec((3, _R, D), lambda i: (0, i, 0)),
            pl.BlockSpec((_R, 3), lambda i: (i, 0)),
        ],
        out_shape=[
            jax.ShapeDtypeStruct((3, N, D), jnp.float32),
            jax.ShapeDtypeStruct((N, 3), jnp.float32),
        ],
    )(x, W1, degs)


def _tc2(acc, g, dinv, b1, W2):
    return pl.pallas_call(
        _tc2_body,
        grid=(N // _R,),
        in_specs=[
            pl.BlockSpec((2, 2, _R, D), lambda i: (0, 0, i, 0)),
            pl.BlockSpec((3, _R, D), lambda i: (0, i, 0)),
            pl.BlockSpec((_R, 3), lambda i: (i, 0)),
            pl.BlockSpec((1, D), lambda i: (0, 0)),
            pl.BlockSpec((D, 3 * D), lambda i: (0, 0)),
        ],
        out_specs=pl.BlockSpec((3, _R, D), lambda i: (0, i, 0)),
        out_shape=jax.ShapeDtypeStruct((3, N, D), jnp.float32),
    )(acc, g, dinv, b1, W2)


def _tc3(acc, g2, dinv, b2, Wc, bc):
    return pl.pallas_call(
        _tc3_body,
        grid=(N // _R,),
        in_specs=[
            pl.BlockSpec((2, 2, _R, D), lambda i: (0, 0, i, 0)),
            pl.BlockSpec((3, _R, D), lambda i: (0, i, 0)),
            pl.BlockSpec((_R, 3), lambda i: (i, 0)),
            pl.BlockSpec((1, D), lambda i: (0, 0)),
            pl.BlockSpec((HALF, 3 * D), lambda i: (0, 0)),
            pl.BlockSpec((1, HALF), lambda i: (0, 0)),
        ],
        out_specs=pl.BlockSpec((_R, HALF), lambda i: (i, 0)),
        out_shape=jax.ShapeDtypeStruct((N, HALF), jnp.float32),
    )(acc, g2, dinv, b2, Wc, bc.reshape(1, HALF))


def kernel(x, edge_index, edge_in, edge_out, in_w, out_w, W1, b1, W2, b2, Wc, bc):
    ei = edge_index.astype(jnp.int32)
    ein = edge_in.astype(jnp.int32)
    eout = edge_out.astype(jnp.int32)
    in_w = in_w.astype(jnp.float32)
    out_w = out_w.astype(jnp.float32)

    padw = PW - 2 * E
    padu = PU - E
    cw = jnp.concatenate([ein[1] + N, eout[1] + 2 * N,
                          jnp.zeros((padw,), jnp.int32)])
    ww = jnp.concatenate([in_w, out_w, jnp.zeros((padw,), jnp.float32)])
    cu = jnp.concatenate([ei[1], jnp.zeros((padu,), jnp.int32)])
    wu = jnp.concatenate([jnp.ones((E,), jnp.float32),
                          jnp.zeros((padu,), jnp.float32)])
    cw = cw.reshape(PW // DCH, DCH)
    ww = ww.reshape(PW // DCH, DCH)
    cu = cu.reshape(PU // DCH, DCH)
    wu = wu.reshape(PU // DCH, DCH)

    pad = EP - E
    zr = jnp.full((pad,), ROWS, jnp.int32)
    zc = jnp.zeros((pad,), jnp.int32)
    zw = jnp.zeros((pad,), jnp.float32)
    r0 = jnp.concatenate([ei[0], zr]).reshape(SET_CHUNKS, CHUNK)
    c0 = jnp.concatenate([ei[1], zc]).reshape(SET_CHUNKS, CHUNK)
    r1 = jnp.concatenate([ein[0] + N, zr]).reshape(SET_CHUNKS, CHUNK)
    c1 = jnp.concatenate([ein[1], zc]).reshape(SET_CHUNKS, CHUNK)
    w1 = jnp.concatenate([in_w, zw]).reshape(SET_CHUNKS, CHUNK)
    r2 = jnp.concatenate([eout[0] + 2 * N, zr]).reshape(SET_CHUNKS, CHUNK)
    c2 = jnp.concatenate([eout[1], zc]).reshape(SET_CHUNKS, CHUNK)
    w2 = jnp.concatenate([out_w, zw]).reshape(SET_CHUNKS, CHUNK)

    _deg_kernel, _spmm_kernel = _sc_kernels()
    dd = _deg_kernel(cw, ww, cu, wu)
    degs = dd[:, :, 0].reshape(2, 3, N).transpose(2, 1, 0)

    g, dinv = _tc1(x, W1, degs)

    def table(garr):
        return jnp.concatenate([garr.reshape(ROWS, D),
                                jnp.zeros((8, D), jnp.float32)])

    acc1 = _spmm_kernel(r0, c0, r1, c1, w1, r2, c2, w2, table(g))
    g2 = _tc2(acc1, g, dinv, b1, W2)
    acc2 = _spmm_kernel(r0, c0, r1, c1, w1, r2, c2, w2, table(g2))

    return _tc3(acc2, g2, dinv, b2, Wc, bc)


# speedup vs baseline: 1.5131x; 1.0630x over previous
import functools

import jax
import jax.numpy as jnp
from jax import lax
from jax.experimental import pallas as pl
from jax.experimental.pallas import tpu as pltpu
from jax.experimental.pallas import tpu_sc as plsc

N = 10000
E = 320000
D = 128
HALF = 64
ROWS = 3 * N
TAB = 2 * ROWS
TABP = TAB + 8
CHUNK = 128
IDXB = 8
DCH = 128
DIDXB = 4

PW = 655360
PU = 327680
DEG_W = PW // 32
DEG_U = PU // 32

EP = 327680
SET_CHUNKS = EP // CHUNK
P1_CHUNKS = SET_CHUNKS // 16
P2A = 96
P2B = 64


def _deg_body(cw_hbm, ww_hbm, cu_hbm, wu_hbm, out_hbm, cbuf, wbuf, dbuf, dacc):
    c = lax.axis_index("c")
    s = lax.axis_index("s")
    wid = c * 16 + s
    zero16 = jnp.zeros((16,), jnp.float32)

    def zfill(i, carry):
        dbuf[i, pl.ds(0, 16)] = zero16
        return carry

    lax.fori_loop(0, DCH, zfill, 0)

    def zcopy(b, carry):
        pltpu.sync_copy(dbuf, dacc.at[pl.ds(s * 1875 + b * DCH, DCH)])
        return carry

    lax.fori_loop(0, 14, zcopy, 0)
    pltpu.sync_copy(dbuf.at[pl.ds(0, 83)],
                    dacc.at[pl.ds(s * 1875 + 14 * DCH, 83)])
    plsc.subcore_barrier()

    def region(col_hbm, w_hbm, base, nchunks):
        def chunk(j, carry):
            jj = j % DIDXB

            @pl.when(jj == 0)
            def _():
                cb = base + j
                pltpu.sync_copy(col_hbm.at[pl.ds(cb, DIDXB)], cbuf)
                pltpu.sync_copy(w_hbm.at[pl.ds(cb, DIDXB)], wbuf)

            def grp(g, carry2):
                wv = wbuf[jj, pl.ds(g * 16, 16)]
                for j16 in range(16):
                    bv = jnp.full((16,), wv[j16], jnp.float32)
                    dbuf[g * 16 + j16, pl.ds(0, 16)] = bv
                return carry2

            lax.fori_loop(0, DCH // 16, grp, 0)
            pltpu.sync_copy(dbuf, dacc.at[cbuf.at[jj]], add=True)
            return carry

        lax.fori_loop(0, nchunks, chunk, 0)

    region(cw_hbm, ww_hbm, wid * (DEG_W // DCH), DEG_W // DCH)
    region(cu_hbm, wu_hbm, wid * (DEG_U // DCH), DEG_U // DCH)
    plsc.subcore_barrier()
    pltpu.sync_copy(dacc.at[pl.ds(s * 1875, 1875)],
                    out_hbm.at[c, pl.ds(s * 1875, 1875)])


def _spmm_body(r0, c0, r1, c1, w1, r2, c2, w2, gtab_hbm, out_hbm,
               rbuf, cbuf, wbuf, gbufA, gbufB, acc, gsA, gsB, ssA, ssB, isem):
    c = lax.axis_index("c")
    s = lax.axis_index("s")
    zero16 = jnp.zeros((16,), jnp.float32)

    def zfill(i, carry):
        for q in range(D // 16):
            gbufA[i, pl.ds(q * 16, 16)] = zero16
        return carry

    def zero_acc():
        lax.fori_loop(0, CHUNK, zfill, 0)

        def zcopy(b, carry):
            pltpu.sync_copy(gbufA, acc.at[pl.ds(s * 625 + b * CHUNK, CHUNK)])
            return carry

        lax.fori_loop(0, 4, zcopy, 0)
        pltpu.sync_copy(gbufA.at[pl.ds(0, 113)],
                        acc.at[pl.ds(s * 625 + 4 * CHUNK, 113)])

    def copy_out(phase):
        pltpu.sync_copy(acc.at[pl.ds(s * 625, 625)],
                        out_hbm.at[c, phase, pl.ds(s * 625, 625)])

    def scale(gbuf, slot, wrow):
        def grp(g, carry2):
            wv = wbuf[slot, wrow, pl.ds(g * 16, 16)]
            for j16 in range(16):
                bv = jnp.full((16,), wv[j16], jnp.float32)
                e = g * 16 + j16
                for q in range(D // 16):
                    v = gbuf[e, pl.ds(q * 16, 16)]
                    gbuf[e, pl.ds(q * 16, 16)] = v * bv
            return carry2

        lax.fori_loop(0, CHUNK // 16, grp, 0)

    def wait_gather(gbuf, sem):
        pltpu.make_async_copy(gtab_hbm.at[rbuf.at[0, 0]], gbuf, sem).wait()

    def wait_scatter(gbuf, sem):
        pltpu.make_async_copy(gbuf, acc.at[cbuf.at[0, 0]], sem).wait()

    def region(rh, ch, wh, base_row, pairs, weighted):
        nblk = (2 * pairs) // IDXB

        def fetch_idx(b, slot):
            row0 = base_row + b * IDXB
            pltpu.async_copy(rh.at[pl.ds(row0, IDXB)], rbuf.at[slot], isem)
            pltpu.async_copy(ch.at[pl.ds(row0, IDXB)], cbuf.at[slot], isem)
            if weighted:
                pltpu.async_copy(wh.at[pl.ds(row0, IDXB)], wbuf.at[slot], isem)

        def wait_idx(slot):
            pltpu.make_async_copy(rh.at[pl.ds(base_row, IDXB)],
                                  rbuf.at[slot], isem).wait()
            pltpu.make_async_copy(ch.at[pl.ds(base_row, IDXB)],
                                  cbuf.at[slot], isem).wait()
            if weighted:
                pltpu.make_async_copy(wh.at[pl.ds(base_row, IDXB)],
                                      wbuf.at[slot], isem).wait()

        fetch_idx(0, 0)

        def pair(j, carry):
            la = (2 * j) % IDXB
            b = (2 * j) // IDXB
            slot = b % 2

            @pl.when(j > 0)
            def _():
                wait_scatter(gbufB, ssB)

            @pl.when(la == 0)
            def _():
                wait_idx(slot)

                @pl.when(b + 1 < nblk)
                def _():
                    fetch_idx(b + 1, 1 - slot)

                pltpu.async_copy(gtab_hbm.at[rbuf.at[slot, la]], gbufA, gsA)

            wait_gather(gbufA, gsA)
            pltpu.async_copy(gtab_hbm.at[rbuf.at[slot, la + 1]], gbufB, gsB)
            if weighted:
                scale(gbufA, slot, la)
            pltpu.async_copy(gbufA, acc.at[cbuf.at[slot, la]], ssA, add=True)

            wait_gather(gbufB, gsB)
            wait_scatter(gbufA, ssA)

            @pl.when((la < IDXB - 2) & (j < pairs - 1))
            def _():
                pltpu.async_copy(gtab_hbm.at[rbuf.at[slot, la + 2]], gbufA, gsA)

            if weighted:
                scale(gbufB, slot, la + 1)
            pltpu.async_copy(gbufB, acc.at[cbuf.at[slot, la + 1]], ssB, add=True)
            return carry

        lax.fori_loop(0, pairs, pair, 0)
        wait_scatter(gbufB, ssB)

    zero_acc()
    plsc.subcore_barrier()

    @pl.when(c == 0)
    def _():
        region(r0, c0, None, s * P1_CHUNKS, P1_CHUNKS // 2, False)

    @pl.when(c == 1)
    def _():
        region(r1, c1, w1, s * P1_CHUNKS, P1_CHUNKS // 2, True)

    plsc.subcore_barrier()
    copy_out(0)
    zero_acc()
    plsc.subcore_barrier()

    @pl.when(c == 0)
    def _():
        region(r2, c2, w2, s * P2A, P2A // 2, True)

    @pl.when(c == 1)
    def _():
        region(r2, c2, w2, 16 * P2A + s * P2B, P2B // 2, True)

    plsc.subcore_barrier()
    copy_out(1)


@functools.cache
def _sc_kernels():
    mesh = plsc.VectorSubcoreMesh(core_axis_name="c", subcore_axis_name="s",
                                  num_cores=2, num_subcores=16)
    params = pltpu.CompilerParams(use_tc_tiling_on_sc=False)
    deg = pl.kernel(
        _deg_body,
        out_type=jax.ShapeDtypeStruct((2, ROWS, 16), jnp.float32),
        mesh=mesh,
        compiler_params=params,
        scratch_types=[
            pltpu.VMEM((DIDXB, DCH), jnp.int32),
            pltpu.VMEM((DIDXB, DCH), jnp.float32),
            pltpu.VMEM((DCH, 16), jnp.float32),
            pltpu.VMEM_SHARED((ROWS, 16), jnp.float32),
        ],
    )
    spmm = pl.kernel(
        _spmm_body,
        out_type=jax.ShapeDtypeStruct((2, 2, N, D), jnp.float32),
        mesh=mesh,
        compiler_params=params,
        scratch_types=[
            pltpu.VMEM((2, IDXB, CHUNK), jnp.int32),
            pltpu.VMEM((2, IDXB, CHUNK), jnp.int32),
            pltpu.VMEM((2, IDXB, CHUNK), jnp.float32),
            pltpu.VMEM((CHUNK, D), jnp.float32),
            pltpu.VMEM((CHUNK, D), jnp.float32),
            pltpu.VMEM_SHARED((N, D), jnp.float32),
            pltpu.SemaphoreType.DMA,
            pltpu.SemaphoreType.DMA,
            pltpu.SemaphoreType.DMA,
            pltpu.SemaphoreType.DMA,
            pltpu.SemaphoreType.DMA,
        ],
    )
    return deg, spmm


_R = 1000


def _tc1_body(x_ref, w1_ref, degs_ref, g_ref, dinv_ref):
    deg = jnp.sum(degs_ref[...], axis=2) + 1.0
    dinv = jnp.where(deg > 0, lax.rsqrt(deg), 0.0)
    dinv_ref[...] = dinv
    h = jnp.dot(x_ref[...], w1_ref[...].T, preferred_element_type=jnp.float32)
    for k in range(3):
        g_ref[k] = h * dinv[:, k][:, None]


def _tc_mid_parts(acc_ref, g_ref, dinv_ref, b_ref):
    accs = [acc_ref[0, 0], acc_ref[1, 0], acc_ref[0, 1] + acc_ref[1, 1]]
    dinv = dinv_ref[...]
    parts = []
    for k in range(3):
        xk = dinv[:, k][:, None] * (accs[k] + g_ref[k]) + b_ref[...]
        parts.append(jax.nn.relu(xk))
    return jnp.concatenate(parts, axis=1)


def _tc2_body(acc_ref, g_ref, dinv_ref, b1_ref, w2_ref, g2_ref):
    xcat = _tc_mid_parts(acc_ref, g_ref, dinv_ref, b1_ref)
    h2 = jnp.dot(xcat, w2_ref[...].T, preferred_element_type=jnp.float32)
    dinv = dinv_ref[...]
    for k in range(3):
        g2_ref[k] = h2 * dinv[:, k][:, None]


def _tc3_body(acc_ref, g2_ref, dinv_ref, b2_ref, wc_ref, bc_ref, out_ref):
    xcat = _tc_mid_parts(acc_ref, g2_ref, dinv_ref, b2_ref)
    logits = jnp.dot(xcat, wc_ref[...].T, preferred_element_type=jnp.float32)
    logits = logits + bc_ref[...]
    m = jnp.max(logits, axis=1, keepdims=True)
    lse = jnp.log(jnp.sum(jnp.exp(logits - m), axis=1, keepdims=True)) + m
    out_ref[...] = logits - lse


def _tc1(x, W1, degs):
    return pl.pallas_call(
        _tc1_body,
        grid=(N // _R,),
        in_specs=[
            pl.BlockSpec((_R, D), lambda i: (i, 0)),
            pl.BlockSpec((D, D), lambda i: (0, 0)),
            pl.BlockSpec((_R, 3, 2), lambda i: (i, 0, 0)),
        ],
        out_specs=[
            pl.BlockSpec((3, _R, D), lambda i: (0, i, 0)),
            pl.BlockSpec((_R, 3), lambda i: (i, 0)),
        ],
        out_shape=[
            jax.ShapeDtypeStruct((3, N, D), jnp.float32),
            jax.ShapeDtypeStruct((N, 3), jnp.float32),
        ],
    )(x, W1, degs)


def _tc2(acc, g, dinv, b1, W2):
    return pl.pallas_call(
        _tc2_body,
        grid=(N // _R,),
        in_specs=[
            pl.BlockSpec((2, 2, _R, D), lambda i: (0, 0, i, 0)),
            pl.BlockSpec((3, _R, D), lambda i: (0, i, 0)),
            pl.BlockSpec((_R, 3), lambda i: (i, 0)),
            pl.BlockSpec((1, D), lambda i: (0, 0)),
            pl.BlockSpec((D, 3 * D), lambda i: (0, 0)),
        ],
        out_specs=pl.BlockSpec((3, _R, D), lambda i: (0, i, 0)),
        out_shape=jax.ShapeDtypeStruct((3, N, D), jnp.float32),
    )(acc, g, dinv, b1, W2)


def _tc3(acc, g2, dinv, b2, Wc, bc):
    return pl.pallas_call(
        _tc3_body,
        grid=(N // _R,),
        in_specs=[
            pl.BlockSpec((2, 2, _R, D), lambda i: (0, 0, i, 0)),
            pl.BlockSpec((3, _R, D), lambda i: (0, i, 0)),
            pl.BlockSpec((_R, 3), lambda i: (i, 0)),
            pl.BlockSpec((1, D), lambda i: (0, 0)),
            pl.BlockSpec((HALF, 3 * D), lambda i: (0, 0)),
            pl.BlockSpec((1, HALF), lambda i: (0, 0)),
        ],
        out_specs=pl.BlockSpec((_R, HALF), lambda i: (i, 0)),
        out_shape=jax.ShapeDtypeStruct((N, HALF), jnp.float32),
    )(acc, g2, dinv, b2, Wc, bc.reshape(1, HALF))


def kernel(x, edge_index, edge_in, edge_out, in_w, out_w, W1, b1, W2, b2, Wc, bc):
    ei = edge_index.astype(jnp.int32)
    ein = edge_in.astype(jnp.int32)
    eout = edge_out.astype(jnp.int32)
    in_w = in_w.astype(jnp.float32)
    out_w = out_w.astype(jnp.float32)

    padw = PW - 2 * E
    padu = PU - E
    cw = jnp.concatenate([ein[1] + N, eout[1] + 2 * N,
                          jnp.zeros((padw,), jnp.int32)])
    ww = jnp.concatenate([in_w, out_w, jnp.zeros((padw,), jnp.float32)])
    cu = jnp.concatenate([ei[1], jnp.zeros((padu,), jnp.int32)])
    wu = jnp.concatenate([jnp.ones((E,), jnp.float32),
                          jnp.zeros((padu,), jnp.float32)])
    cw = cw.reshape(PW // DCH, DCH)
    ww = ww.reshape(PW // DCH, DCH)
    cu = cu.reshape(PU // DCH, DCH)
    wu = wu.reshape(PU // DCH, DCH)

    pad = EP - E
    zr = jnp.full((pad,), ROWS, jnp.int32)
    zc = jnp.zeros((pad,), jnp.int32)
    zw = jnp.zeros((pad,), jnp.float32)
    r0 = jnp.concatenate([ei[0], zr]).reshape(SET_CHUNKS, CHUNK)
    c0 = jnp.concatenate([ei[1], zc]).reshape(SET_CHUNKS, CHUNK)
    r1 = jnp.concatenate([ein[0] + N, zr]).reshape(SET_CHUNKS, CHUNK)
    c1 = jnp.concatenate([ein[1], zc]).reshape(SET_CHUNKS, CHUNK)
    w1 = jnp.concatenate([in_w, zw]).reshape(SET_CHUNKS, CHUNK)
    r2 = jnp.concatenate([eout[0] + 2 * N, zr]).reshape(SET_CHUNKS, CHUNK)
    c2 = jnp.concatenate([eout[1], zc]).reshape(SET_CHUNKS, CHUNK)
    w2 = jnp.concatenate([out_w, zw]).reshape(SET_CHUNKS, CHUNK)

    _deg_kernel, _spmm_kernel = _sc_kernels()
    dd = _deg_kernel(cw, ww, cu, wu)
    degs = dd[:, :, 0].reshape(2, 3, N).transpose(2, 1, 0)

    g, dinv = _tc1(x, W1, degs)

    def table(garr):
        return jnp.concatenate([garr.reshape(ROWS, D),
                                jnp.zeros((8, D), jnp.float32)])

    acc1 = _spmm_kernel(r0, c0, r1, c1, w1, r2, c2, w2, table(g))
    g2 = _tc2(acc1, g, dinv, b1, W2)
    acc2 = _spmm_kernel(r0, c0, r1, c1, w1, r2, c2, w2, table(g2))

    return _tc3(acc2, g2, dinv, b2, Wc, bc)


# speedup vs baseline: 1.5647x; 1.0341x over previous
import functools

import jax
import jax.numpy as jnp
from jax import lax
from jax.experimental import pallas as pl
from jax.experimental.pallas import tpu as pltpu
from jax.experimental.pallas import tpu_sc as plsc

N = 10000
E = 320000
D = 128
HALF = 64
ROWS = 3 * N
TAB = 2 * ROWS
TABP = TAB + 8
CHUNK = 128
IDXB = 8
DCH = 128
DIDXB = 4

PW = 655360
PU = 327680
DEG_W = PW // 32
DEG_U = PU // 32

EP = 327680
SET_CHUNKS = EP // CHUNK
P1_CHUNKS = SET_CHUNKS // 16
P2A = 128
P2B = 32


def _deg_body(cw_hbm, ww_hbm, cu_hbm, wu_hbm, out_hbm, cbuf, wbuf, dbuf, dacc):
    c = lax.axis_index("c")
    s = lax.axis_index("s")
    wid = c * 16 + s
    zero16 = jnp.zeros((16,), jnp.float32)

    def zfill(i, carry):
        dbuf[i, pl.ds(0, 16)] = zero16
        return carry

    lax.fori_loop(0, DCH, zfill, 0)

    def zcopy(b, carry):
        pltpu.sync_copy(dbuf, dacc.at[pl.ds(s * 1875 + b * DCH, DCH)])
        return carry

    lax.fori_loop(0, 14, zcopy, 0)
    pltpu.sync_copy(dbuf.at[pl.ds(0, 83)],
                    dacc.at[pl.ds(s * 1875 + 14 * DCH, 83)])
    plsc.subcore_barrier()

    def region(col_hbm, w_hbm, base, nchunks):
        def chunk(j, carry):
            jj = j % DIDXB

            @pl.when(jj == 0)
            def _():
                cb = base + j
                pltpu.sync_copy(col_hbm.at[pl.ds(cb, DIDXB)], cbuf)
                pltpu.sync_copy(w_hbm.at[pl.ds(cb, DIDXB)], wbuf)

            def grp(g, carry2):
                wv = wbuf[jj, pl.ds(g * 16, 16)]
                for j16 in range(16):
                    bv = jnp.full((16,), wv[j16], jnp.float32)
                    dbuf[g * 16 + j16, pl.ds(0, 16)] = bv
                return carry2

            lax.fori_loop(0, DCH // 16, grp, 0)
            pltpu.sync_copy(dbuf, dacc.at[cbuf.at[jj]], add=True)
            return carry

        lax.fori_loop(0, nchunks, chunk, 0)

    region(cw_hbm, ww_hbm, wid * (DEG_W // DCH), DEG_W // DCH)
    region(cu_hbm, wu_hbm, wid * (DEG_U // DCH), DEG_U // DCH)
    plsc.subcore_barrier()
    pltpu.sync_copy(dacc.at[pl.ds(s * 1875, 1875)],
                    out_hbm.at[c, pl.ds(s * 1875, 1875)])


def _spmm_body(r0, c0, r1, c1, w1, r2, c2, w2, gtab_hbm, out_hbm,
               rbuf, cbuf, wbuf, gbufA, gbufB, acc, gsA, gsB, ssA, ssB, isem):
    c = lax.axis_index("c")
    s = lax.axis_index("s")
    zero16 = jnp.zeros((16,), jnp.float32)

    def zfill(i, carry):
        for q in range(D // 16):
            gbufA[i, pl.ds(q * 16, 16)] = zero16
        return carry

    def zero_acc():
        lax.fori_loop(0, CHUNK, zfill, 0)

        def zcopy(b, carry):
            pltpu.sync_copy(gbufA, acc.at[pl.ds(s * 625 + b * CHUNK, CHUNK)])
            return carry

        lax.fori_loop(0, 4, zcopy, 0)
        pltpu.sync_copy(gbufA.at[pl.ds(0, 113)],
                        acc.at[pl.ds(s * 625 + 4 * CHUNK, 113)])

    def copy_out(phase):
        pltpu.sync_copy(acc.at[pl.ds(s * 625, 625)],
                        out_hbm.at[c, phase, pl.ds(s * 625, 625)])

    def scale(gbuf, slot, wrow):
        def grp(g, carry2):
            wv = wbuf[slot, wrow, pl.ds(g * 16, 16)]
            bvs = [jnp.full((16,), wv[j16], jnp.float32) for j16 in range(16)]
            for q in range(D // 16):
                for j16 in range(16):
                    e = g * 16 + j16
                    v = gbuf[e, pl.ds(q * 16, 16)]
                    gbuf[e, pl.ds(q * 16, 16)] = v * bvs[j16]
            return carry2

        lax.fori_loop(0, CHUNK // 16, grp, 0)

    def wait_gather(gbuf, sem):
        pltpu.make_async_copy(gtab_hbm.at[rbuf.at[0, 0]], gbuf, sem).wait()

    def wait_scatter(gbuf, sem):
        pltpu.make_async_copy(gbuf, acc.at[cbuf.at[0, 0]], sem).wait()

    def region(rh, ch, wh, base_row, pairs, weighted):
        nblk = (2 * pairs) // IDXB

        def fetch_idx(b, slot):
            row0 = base_row + b * IDXB
            pltpu.async_copy(rh.at[pl.ds(row0, IDXB)], rbuf.at[slot], isem)
            pltpu.async_copy(ch.at[pl.ds(row0, IDXB)], cbuf.at[slot], isem)
            if weighted:
                pltpu.async_copy(wh.at[pl.ds(row0, IDXB)], wbuf.at[slot], isem)

        def wait_idx(slot):
            pltpu.make_async_copy(rh.at[pl.ds(base_row, IDXB)],
                                  rbuf.at[slot], isem).wait()
            pltpu.make_async_copy(ch.at[pl.ds(base_row, IDXB)],
                                  cbuf.at[slot], isem).wait()
            if weighted:
                pltpu.make_async_copy(wh.at[pl.ds(base_row, IDXB)],
                                      wbuf.at[slot], isem).wait()

        fetch_idx(0, 0)

        def pair(j, carry):
            la = (2 * j) % IDXB
            b = (2 * j) // IDXB
            slot = b % 2

            @pl.when(j > 0)
            def _():
                wait_scatter(gbufB, ssB)

            @pl.when(la == 0)
            def _():
                wait_idx(slot)

                @pl.when(b + 1 < nblk)
                def _():
                    fetch_idx(b + 1, 1 - slot)

                pltpu.async_copy(gtab_hbm.at[rbuf.at[slot, la]], gbufA, gsA)

            pltpu.async_copy(gtab_hbm.at[rbuf.at[slot, la + 1]], gbufB, gsB)
            wait_gather(gbufA, gsA)
            if weighted:
                scale(gbufA, slot, la)
            pltpu.async_copy(gbufA, acc.at[cbuf.at[slot, la]], ssA, add=True)

            wait_gather(gbufB, gsB)
            wait_scatter(gbufA, ssA)

            @pl.when((la < IDXB - 2) & (j < pairs - 1))
            def _():
                pltpu.async_copy(gtab_hbm.at[rbuf.at[slot, la + 2]], gbufA, gsA)

            if weighted:
                scale(gbufB, slot, la + 1)
            pltpu.async_copy(gbufB, acc.at[cbuf.at[slot, la + 1]], ssB, add=True)
            return carry

        lax.fori_loop(0, pairs, pair, 0)
        wait_scatter(gbufB, ssB)

    zero_acc()
    plsc.subcore_barrier()

    @pl.when(c == 0)
    def _():
        region(r0, c0, None, s * P1_CHUNKS, P1_CHUNKS // 2, False)

    @pl.when(c == 1)
    def _():
        region(r1, c1, w1, s * P1_CHUNKS, P1_CHUNKS // 2, True)

    plsc.subcore_barrier()
    copy_out(0)
    zero_acc()
    plsc.subcore_barrier()

    @pl.when(c == 0)
    def _():
        region(r2, c2, w2, s * P2A, P2A // 2, True)

    @pl.when(c == 1)
    def _():
        region(r2, c2, w2, 16 * P2A + s * P2B, P2B // 2, True)

    plsc.subcore_barrier()
    copy_out(1)


@functools.cache
def _sc_kernels():
    mesh = plsc.VectorSubcoreMesh(core_axis_name="c", subcore_axis_name="s",
                                  num_cores=2, num_subcores=16)
    params = pltpu.CompilerParams(use_tc_tiling_on_sc=False)
    deg = pl.kernel(
        _deg_body,
        out_type=jax.ShapeDtypeStruct((2, ROWS, 16), jnp.float32),
        mesh=mesh,
        compiler_params=params,
        scratch_types=[
            pltpu.VMEM((DIDXB, DCH), jnp.int32),
            pltpu.VMEM((DIDXB, DCH), jnp.float32),
            pltpu.VMEM((DCH, 16), jnp.float32),
            pltpu.VMEM_SHARED((ROWS, 16), jnp.float32),
        ],
    )
    spmm = pl.kernel(
        _spmm_body,
        out_type=jax.ShapeDtypeStruct((2, 2, N, D), jnp.float32),
        mesh=mesh,
        compiler_params=params,
        scratch_types=[
            pltpu.VMEM((2, IDXB, CHUNK), jnp.int32),
            pltpu.VMEM((2, IDXB, CHUNK), jnp.int32),
            pltpu.VMEM((2, IDXB, CHUNK), jnp.float32),
            pltpu.VMEM((CHUNK, D), jnp.float32),
            pltpu.VMEM((CHUNK, D), jnp.float32),
            pltpu.VMEM_SHARED((N, D), jnp.float32),
            pltpu.SemaphoreType.DMA,
            pltpu.SemaphoreType.DMA,
            pltpu.SemaphoreType.DMA,
            pltpu.SemaphoreType.DMA,
            pltpu.SemaphoreType.DMA,
        ],
    )
    return deg, spmm


_R = 1000


def _tc1_body(x_ref, w1_ref, degs_ref, g_ref, dinv_ref):
    deg = jnp.sum(degs_ref[...], axis=2) + 1.0
    dinv = jnp.where(deg > 0, lax.rsqrt(deg), 0.0)
    dinv_ref[...] = dinv
    h = jnp.dot(x_ref[...], w1_ref[...].T, preferred_element_type=jnp.float32)
    for k in range(3):
        g_ref[k] = h * dinv[:, k][:, None]


def _tc_mid_parts(acc_ref, g_ref, dinv_ref, b_ref):
    accs = [acc_ref[0, 0], acc_ref[1, 0], acc_ref[0, 1] + acc_ref[1, 1]]
    dinv = dinv_ref[...]
    parts = []
    for k in range(3):
        xk = dinv[:, k][:, None] * (accs[k] + g_ref[k]) + b_ref[...]
        parts.append(jax.nn.relu(xk))
    return jnp.concatenate(parts, axis=1)


def _tc2_body(acc_ref, g_ref, dinv_ref, b1_ref, w2_ref, g2_ref):
    xcat = _tc_mid_parts(acc_ref, g_ref, dinv_ref, b1_ref)
    h2 = jnp.dot(xcat, w2_ref[...].T, preferred_element_type=jnp.float32)
    dinv = dinv_ref[...]
    for k in range(3):
        g2_ref[k] = h2 * dinv[:, k][:, None]


def _tc3_body(acc_ref, g2_ref, dinv_ref, b2_ref, wc_ref, bc_ref, out_ref):
    xcat = _tc_mid_parts(acc_ref, g2_ref, dinv_ref, b2_ref)
    logits = jnp.dot(xcat, wc_ref[...].T, preferred_element_type=jnp.float32)
    logits = logits + bc_ref[...]
    m = jnp.max(logits, axis=1, keepdims=True)
    lse = jnp.log(jnp.sum(jnp.exp(logits - m), axis=1, keepdims=True)) + m
    out_ref[...] = logits - lse


def _tc1(x, W1, degs):
    return pl.pallas_call(
        _tc1_body,
        grid=(N // _R,),
        in_specs=[
            pl.BlockSpec((_R, D), lambda i: (i, 0)),
            pl.BlockSpec((D, D), lambda i: (0, 0)),
            pl.BlockSpec((_R, 3, 2), lambda i: (i, 0, 0)),
        ],
        out_specs=[
            pl.BlockSpec((3, _R, D), lambda i: (0, i, 0)),
            pl.BlockSpec((_R, 3), lambda i: (i, 0)),
        ],
        out_shape=[
            jax.ShapeDtypeStruct((3, N, D), jnp.float32),
            jax.ShapeDtypeStruct((N, 3), jnp.float32),
        ],
    )(x, W1, degs)


def _tc2(acc, g, dinv, b1, W2):
    return pl.pallas_call(
        _tc2_body,
        grid=(N // _R,),
        in_specs=[
            pl.BlockSpec((2, 2, _R, D), lambda i: (0, 0, i, 0)),
            pl.BlockSpec((3, _R, D), lambda i: (0, i, 0)),
            pl.BlockSpec((_R, 3), lambda i: (i, 0)),
            pl.BlockSpec((1, D), lambda i: (0, 0)),
            pl.BlockSpec((D, 3 * D), lambda i: (0, 0)),
        ],
        out_specs=pl.BlockSpec((3, _R, D), lambda i: (0, i, 0)),
        out_shape=jax.ShapeDtypeStruct((3, N, D), jnp.float32),
    )(acc, g, dinv, b1, W2)


def _tc3(acc, g2, dinv, b2, Wc, bc):
    return pl.pallas_call(
        _tc3_body,
        grid=(N // _R,),
        in_specs=[
            pl.BlockSpec((2, 2, _R, D), lambda i: (0, 0, i, 0)),
            pl.BlockSpec((3, _R, D), lambda i: (0, i, 0)),
            pl.BlockSpec((_R, 3), lambda i: (i, 0)),
            pl.BlockSpec((1, D), lambda i: (0, 0)),
            pl.BlockSpec((HALF, 3 * D), lambda i: (0, 0)),
            pl.BlockSpec((1, HALF), lambda i: (0, 0)),
        ],
        out_specs=pl.BlockSpec((_R, HALF), lambda i: (i, 0)),
        out_shape=jax.ShapeDtypeStruct((N, HALF), jnp.float32),
    )(acc, g2, dinv, b2, Wc, bc.reshape(1, HALF))


def kernel(x, edge_index, edge_in, edge_out, in_w, out_w, W1, b1, W2, b2, Wc, bc):
    ei = edge_index.astype(jnp.int32)
    ein = edge_in.astype(jnp.int32)
    eout = edge_out.astype(jnp.int32)
    in_w = in_w.astype(jnp.float32)
    out_w = out_w.astype(jnp.float32)

    padw = PW - 2 * E
    padu = PU - E
    cw = jnp.concatenate([ein[1] + N, eout[1] + 2 * N,
                          jnp.zeros((padw,), jnp.int32)])
    ww = jnp.concatenate([in_w, out_w, jnp.zeros((padw,), jnp.float32)])
    cu = jnp.concatenate([ei[1], jnp.zeros((padu,), jnp.int32)])
    wu = jnp.concatenate([jnp.ones((E,), jnp.float32),
                          jnp.zeros((padu,), jnp.float32)])
    cw = cw.reshape(PW // DCH, DCH)
    ww = ww.reshape(PW // DCH, DCH)
    cu = cu.reshape(PU // DCH, DCH)
    wu = wu.reshape(PU // DCH, DCH)

    pad = EP - E
    zr = jnp.full((pad,), ROWS, jnp.int32)
    zc = jnp.zeros((pad,), jnp.int32)
    zw = jnp.zeros((pad,), jnp.float32)
    r0 = jnp.concatenate([ei[0], zr]).reshape(SET_CHUNKS, CHUNK)
    c0 = jnp.concatenate([ei[1], zc]).reshape(SET_CHUNKS, CHUNK)
    r1 = jnp.concatenate([ein[0] + N, zr]).reshape(SET_CHUNKS, CHUNK)
    c1 = jnp.concatenate([ein[1], zc]).reshape(SET_CHUNKS, CHUNK)
    w1 = jnp.concatenate([in_w, zw]).reshape(SET_CHUNKS, CHUNK)
    r2 = jnp.concatenate([eout[0] + 2 * N, zr]).reshape(SET_CHUNKS, CHUNK)
    c2 = jnp.concatenate([eout[1], zc]).reshape(SET_CHUNKS, CHUNK)
    w2 = jnp.concatenate([out_w, zw]).reshape(SET_CHUNKS, CHUNK)

    _deg_kernel, _spmm_kernel = _sc_kernels()
    dd = _deg_kernel(cw, ww, cu, wu)
    degs = dd[:, :, 0].reshape(2, 3, N).transpose(2, 1, 0)

    g, dinv = _tc1(x, W1, degs)

    def table(garr):
        return jnp.concatenate([garr.reshape(ROWS, D),
                                jnp.zeros((8, D), jnp.float32)])

    acc1 = _spmm_kernel(r0, c0, r1, c1, w1, r2, c2, w2, table(g))
    g2 = _tc2(acc1, g, dinv, b1, W2)
    acc2 = _spmm_kernel(r0, c0, r1, c1, w1, r2, c2, w2, table(g2))

    return _tc3(acc2, g2, dinv, b2, Wc, bc)
